# Initial kernel scaffold; baseline (speedup 1.0000x reference)
import jax
import jax.numpy as jnp
from jax.experimental import pallas as pl


def kernel(in_feat, coord, edge_feat, edge_index, We1, be1, We2, be2, Wc1, bc1, Wc2, Wn1, bn1, Wn2, bn2, Wl, bl):
    raise NotImplementedError("write your pallas kernel here")


# speedup vs baseline: 5.1939x; 5.1939x over previous
import functools

import jax
import jax.numpy as jnp
from jax import lax
from jax.experimental import pallas as pl
from jax.experimental.pallas import tpu as pltpu
from jax.experimental.pallas import tpu_sc as plsc

N = 10000
E = 320000
F = 128
H = 64
W = 128
NC, NS = 2, 16
NW = NC * NS
EPW = E // NW
CB = 80
CH = EPW // CB
NPAD = 10240
STR = NPAD // NS
EB = 2000


def _silu(x):
    return x / (1.0 + jnp.exp(-x))


def _prep_body(infeat_ref, coord_ref, whs_ref, whd_ref, be1_ref, ps_ref, qd_ref):
    x = infeat_ref[:]
    c = coord_ref[:]
    z = jnp.zeros((x.shape[0], W - H - 2), jnp.float32)
    p = jnp.dot(x, whs_ref[:], preferred_element_type=jnp.float32)
    q = jnp.dot(x, whd_ref[:], preferred_element_type=jnp.float32) + be1_ref[:]
    ps_ref[:] = jnp.concatenate([p, c, z], axis=1)
    qd_ref[:] = jnp.concatenate([q, -c, z], axis=1)


def _prep(in_feat, coord, whs, whd, be1):
    return pl.pallas_call(
        _prep_body,
        out_shape=[
            jax.ShapeDtypeStruct((N, W), jnp.float32),
            jax.ShapeDtypeStruct((N, W), jnp.float32),
        ],
    )(in_feat, coord, whs, whd, be1)


def _gather_body(ps, qd, srcw, dstw, gs_out, gd_out, idx_s, idx_d, bufs, bufd,
                 sems, semd):
    c = lax.axis_index("c")
    s = lax.axis_index("s")
    wid = s * NC + c
    pltpu.sync_copy(srcw.at[wid], idx_s)
    pltpu.sync_copy(dstw.at[wid], idx_d)
    base = wid * EPW

    def body(j, carry):
        pltpu.async_copy(ps.at[idx_s.at[j]], bufs, sems).wait()
        pltpu.async_copy(qd.at[idx_d.at[j]], bufd, semd).wait()
        pltpu.sync_copy(bufs, gs_out.at[pl.ds(base + j * CB, CB)])
        pltpu.sync_copy(bufd, gd_out.at[pl.ds(base + j * CB, CB)])
        return carry

    lax.fori_loop(0, CH, body, 0)


def _gather(ps, qd, srcw, dstw):
    fn = functools.partial(
        pl.kernel,
        out_type=[
            jax.ShapeDtypeStruct((E, W), jnp.float32),
            jax.ShapeDtypeStruct((E, W), jnp.float32),
        ],
        mesh=plsc.VectorSubcoreMesh(core_axis_name="c", subcore_axis_name="s"),
        scratch_types=[
            pltpu.VMEM((CH, CB), jnp.int32),
            pltpu.VMEM((CH, CB), jnp.int32),
            pltpu.VMEM((CB, W), jnp.float32),
            pltpu.VMEM((CB, W), jnp.float32),
            pltpu.SemaphoreType.DMA,
            pltpu.SemaphoreType.DMA,
        ],
    )(_gather_body)
    return fn(ps, qd, srcw, dstw)


def _edge_body(gs, gd, ef, wr, we, we2, be2, wc1, bc1, wc2, out):
    srow = gs[:] + gd[:]
    t = srow[:, 0:H]
    d = srow[:, H:H + 2]
    radial = jnp.sum(d * d, axis=1, keepdims=True)
    xdn = d / (jnp.sqrt(radial) + 1e-30)
    t1 = t + radial * wr[:] + jnp.dot(ef[:], we[:],
                                      preferred_element_type=jnp.float32)
    m = _silu(t1)
    m = _silu(jnp.dot(m, we2[:], preferred_element_type=jnp.float32) + be2[:])
    cmid = _silu(jnp.dot(m, wc1[:], preferred_element_type=jnp.float32) + bc1[:])
    cg = jnp.dot(cmid, wc2[:], preferred_element_type=jnp.float32)
    pad = jnp.zeros((m.shape[0], W - H - 2), jnp.float32)
    out[:] = jnp.concatenate([m, cg * xdn, pad], axis=1)


def _edge(gs, gd, edge_feat, wr, we, we2, be2, wc1, bc1, wc2):
    grid = (E // EB,)
    full = lambda shape: pl.BlockSpec(shape, lambda i: (0, 0))
    return pl.pallas_call(
        _edge_body,
        grid=grid,
        in_specs=[
            pl.BlockSpec((EB, W), lambda i: (i, 0)),
            pl.BlockSpec((EB, W), lambda i: (i, 0)),
            pl.BlockSpec((EB, 16), lambda i: (i, 0)),
            full((1, H)),
            full((16, H)),
            full((H, H)),
            full((1, H)),
            full((H, H)),
            full((1, H)),
            full((H, 1)),
        ],
        out_specs=pl.BlockSpec((EB, W), lambda i: (i, 0)),
        out_shape=jax.ShapeDtypeStruct((E, W), jnp.float32),
    )(gs, gd, edge_feat, wr, we, we2, be2, wc1, bc1, wc2)


def _scatter_body(msg, dstw, zrows, acc_out, idx_d, buf, acc_sh):
    c = lax.axis_index("c")
    s = lax.axis_index("s")
    wid = s * NC + c
    pltpu.sync_copy(dstw.at[wid], idx_d)
    pltpu.sync_copy(zrows.at[pl.ds(s * STR, STR)],
                    acc_sh.at[pl.ds(s * STR, STR)])
    plsc.subcore_barrier()
    base = wid * EPW

    def body(j, carry):
        pltpu.sync_copy(msg.at[pl.ds(base + j * CB, CB)], buf)
        pltpu.sync_copy(buf, acc_sh.at[idx_d.at[j]], add=True)
        return carry

    lax.fori_loop(0, CH, body, 0)
    plsc.subcore_barrier()
    pltpu.sync_copy(acc_sh.at[pl.ds(s * STR, STR)],
                    acc_out.at[c].at[pl.ds(s * STR, STR)])


def _scatter(msg, dstw, zrows):
    fn = functools.partial(
        pl.kernel,
        out_type=jax.ShapeDtypeStruct((NC, NPAD, W), jnp.float32),
        mesh=plsc.VectorSubcoreMesh(core_axis_name="c", subcore_axis_name="s"),
        scratch_types=[
            pltpu.VMEM((CH, CB), jnp.int32),
            pltpu.VMEM((CB, W), jnp.float32),
            pltpu.VMEM_SHARED((NPAD, W), jnp.float32),
        ],
    )(_scatter_body)
    return fn(msg, dstw, zrows)


def _node_body(infeat, coord, a0, a1, wn1a, wn1b, bn1, wn2, bn2, wl, bl, out):
    a = a0[:] + a1[:]
    hn = a[:, 0:H]
    xn = a[:, H:H + 2]
    h1 = _silu(jnp.dot(infeat[:], wn1a[:], preferred_element_type=jnp.float32)
               + jnp.dot(hn, wn1b[:], preferred_element_type=jnp.float32)
               + bn1[:])
    h2 = jnp.dot(h1, wn2[:], preferred_element_type=jnp.float32) + bn2[:]
    h3 = jnp.dot(h2, wl[:], preferred_element_type=jnp.float32) + bl[:]
    out[:] = jnp.concatenate([coord[:] + xn, h3], axis=1)


def _node(in_feat, coord, a0, a1, wn1a, wn1b, bn1, wn2, bn2, wl, bl):
    return pl.pallas_call(
        _node_body,
        out_shape=jax.ShapeDtypeStruct((N, 42), jnp.float32),
    )(in_feat, coord, a0, a1, wn1a, wn1b, bn1, wn2, bn2, wl, bl)


def kernel(in_feat, coord, edge_feat, edge_index, We1, be1, We2, be2, Wc1,
           bc1, Wc2, Wn1, bn1, Wn2, bn2, Wl, bl):
    srcw = edge_index[0].reshape(NW, CH, CB)
    dstw = edge_index[1].reshape(NW, CH, CB)

    ps, qd = _prep(in_feat, coord, We1[:F], We1[F:2 * F], be1.reshape(1, H))
    gs, gd = _gather(ps, qd, srcw, dstw)
    msg = _edge(gs, gd, edge_feat, We1[2 * F:2 * F + 1], We1[2 * F + 1:],
                We2, be2.reshape(1, H), Wc1, bc1.reshape(1, H), Wc2)
    zrows = jnp.zeros((NPAD, W), jnp.float32)
    acc = _scatter(msg, dstw, zrows)
    out = _node(in_feat, coord, acc[0, :N], acc[1, :N], Wn1[:F], Wn1[F:],
                bn1.reshape(1, H), Wn2, bn2.reshape(1, H), Wl,
                bl.reshape(1, -1))
    return out


# speedup vs baseline: 6.6307x; 1.2766x over previous
import functools

import jax
import jax.numpy as jnp
from jax import lax
from jax.experimental import pallas as pl
from jax.experimental.pallas import tpu as pltpu
from jax.experimental.pallas import tpu_sc as plsc

N = 10000
E = 320000
F = 128
H = 64
W = 128
NC, NS = 2, 16
NW = NC * NS
EPW = E // NW
CB = 80
CH = EPW // CB
NPAD = 10240
STR = NPAD // NS
EB = 2000


def _silu(x):
    return x / (1.0 + jnp.exp(-x))


def _prep_body(infeat_ref, coord_ref, whs_ref, whd_ref, be1_ref, ps_ref, qd_ref):
    x = infeat_ref[:]
    c = coord_ref[:]
    z = jnp.zeros((x.shape[0], W - H - 2), jnp.float32)
    p = jnp.dot(x, whs_ref[:], preferred_element_type=jnp.float32)
    q = jnp.dot(x, whd_ref[:], preferred_element_type=jnp.float32) + be1_ref[:]
    ps_ref[:] = jnp.concatenate([p, c, z], axis=1)
    qd_ref[:] = jnp.concatenate([q, -c, z], axis=1)


def _prep(in_feat, coord, whs, whd, be1):
    return pl.pallas_call(
        _prep_body,
        out_shape=[
            jax.ShapeDtypeStruct((N, W), jnp.float32),
            jax.ShapeDtypeStruct((N, W), jnp.float32),
        ],
    )(in_feat, coord, whs, whd, be1)


RD = 5
CHO = CH // RD
RDS = 2
CBS = 40
CHS = EPW // CBS


def _gather_body(ps, qd, srcw, dstw, gsum_out, idx_s, idx_d, bufs,
                 sema, semb, semc):
    c = lax.axis_index("c")
    s = lax.axis_index("s")
    wid = s * NC + c
    pltpu.sync_copy(srcw.at[wid], idx_s)
    pltpu.sync_copy(dstw.at[wid], idx_d)
    base = wid * EPW

    def outer(o, carry):
        j0 = o * RD
        da = [pltpu.async_copy(ps.at[idx_s.at[j0 + r]], bufs.at[r], sema)
              for r in range(RD)]
        for d in da:
            d.wait()
        db = [pltpu.async_copy(qd.at[idx_d.at[j0 + r]], bufs.at[r], semb,
                               add=True) for r in range(RD)]
        for d in db:
            d.wait()
        dc = [pltpu.async_copy(bufs.at[r],
                               gsum_out.at[pl.ds(base + (j0 + r) * CB, CB)],
                               semc) for r in range(RD)]
        for d in dc:
            d.wait()
        return carry

    lax.fori_loop(0, CHO, outer, 0)


def _gather(ps, qd, srcw, dstw):
    fn = functools.partial(
        pl.kernel,
        out_type=jax.ShapeDtypeStruct((E, W), jnp.float32),
        mesh=plsc.VectorSubcoreMesh(core_axis_name="c", subcore_axis_name="s"),
        scratch_types=[
            pltpu.VMEM((CH, CB), jnp.int32),
            pltpu.VMEM((CH, CB), jnp.int32),
            pltpu.VMEM((RD, CB, W), jnp.float32),
            pltpu.SemaphoreType.DMA,
            pltpu.SemaphoreType.DMA,
            pltpu.SemaphoreType.DMA,
        ],
    )(_gather_body)
    return fn(ps, qd, srcw, dstw)


def _edge_body(gsum, ef, wr, we, we2, be2, wc1, bc1, wc2, out):
    srow = gsum[:]
    t = srow[:, 0:H]
    d = srow[:, H:H + 2]
    radial = jnp.sum(d * d, axis=1, keepdims=True)
    xdn = d / (jnp.sqrt(radial) + 1e-30)
    t1 = t + radial * wr[:] + jnp.dot(ef[:], we[:],
                                      preferred_element_type=jnp.float32)
    m = _silu(t1)
    m = _silu(jnp.dot(m, we2[:], preferred_element_type=jnp.float32) + be2[:])
    cmid = _silu(jnp.dot(m, wc1[:], preferred_element_type=jnp.float32) + bc1[:])
    cg = jnp.dot(cmid, wc2[:], preferred_element_type=jnp.float32)
    pad = jnp.zeros((m.shape[0], W - H - 2), jnp.float32)
    out[:] = jnp.concatenate([m, cg * xdn, pad], axis=1)


def _edge(gsum, edge_feat, wr, we, we2, be2, wc1, bc1, wc2):
    grid = (E // EB,)
    full = lambda shape: pl.BlockSpec(shape, lambda i: (0, 0))
    return pl.pallas_call(
        _edge_body,
        grid=grid,
        in_specs=[
            pl.BlockSpec((EB, W), lambda i: (i, 0)),
            pl.BlockSpec((EB, 16), lambda i: (i, 0)),
            full((1, H)),
            full((16, H)),
            full((H, H)),
            full((1, H)),
            full((H, H)),
            full((1, H)),
            full((H, 1)),
        ],
        out_specs=pl.BlockSpec((EB, W), lambda i: (i, 0)),
        out_shape=jax.ShapeDtypeStruct((E, W), jnp.float32),
    )(gsum, edge_feat, wr, we, we2, be2, wc1, bc1, wc2)


def _scatter_body(msg, dstw, zrows, acc_out, idx_d, bufs, sema, semb, acc_sh):
    c = lax.axis_index("c")
    s = lax.axis_index("s")
    wid = s * NC + c
    pltpu.sync_copy(dstw.at[wid], idx_d)
    pltpu.sync_copy(zrows.at[pl.ds(s * STR, STR)],
                    acc_sh.at[pl.ds(s * STR, STR)])
    plsc.subcore_barrier()
    base = wid * EPW

    def outer(o, carry):
        j0 = o * RDS
        da = [pltpu.async_copy(msg.at[pl.ds(base + (j0 + r) * CBS, CBS)],
                               bufs.at[r], sema) for r in range(RDS)]
        for d in da:
            d.wait()
        db = [pltpu.async_copy(bufs.at[r], acc_sh.at[idx_d.at[j0 + r]], semb,
                               add=True) for r in range(RDS)]
        for d in db:
            d.wait()
        return carry

    lax.fori_loop(0, CHS // RDS, outer, 0)
    plsc.subcore_barrier()
    pltpu.sync_copy(acc_sh.at[pl.ds(s * STR, STR)],
                    acc_out.at[c].at[pl.ds(s * STR, STR)])


def _scatter(msg, dstw, zrows):
    fn = functools.partial(
        pl.kernel,
        out_type=jax.ShapeDtypeStruct((NC, NPAD, W), jnp.float32),
        mesh=plsc.VectorSubcoreMesh(core_axis_name="c", subcore_axis_name="s"),
        scratch_types=[
            pltpu.VMEM((CHS, CBS), jnp.int32),
            pltpu.VMEM((RDS, CBS, W), jnp.float32),
            pltpu.SemaphoreType.DMA,
            pltpu.SemaphoreType.DMA,
            pltpu.VMEM_SHARED((NPAD, W), jnp.float32),
        ],
    )(_scatter_body)
    return fn(msg, dstw, zrows)


def _node_body(infeat, coord, a0, a1, wn1a, wn1b, bn1, wn2, bn2, wl, bl, out):
    a = a0[:] + a1[:]
    hn = a[:, 0:H]
    xn = a[:, H:H + 2]
    h1 = _silu(jnp.dot(infeat[:], wn1a[:], preferred_element_type=jnp.float32)
               + jnp.dot(hn, wn1b[:], preferred_element_type=jnp.float32)
               + bn1[:])
    h2 = jnp.dot(h1, wn2[:], preferred_element_type=jnp.float32) + bn2[:]
    h3 = jnp.dot(h2, wl[:], preferred_element_type=jnp.float32) + bl[:]
    out[:] = jnp.concatenate([coord[:] + xn, h3], axis=1)


def _node(in_feat, coord, a0, a1, wn1a, wn1b, bn1, wn2, bn2, wl, bl):
    return pl.pallas_call(
        _node_body,
        out_shape=jax.ShapeDtypeStruct((N, 42), jnp.float32),
    )(in_feat, coord, a0, a1, wn1a, wn1b, bn1, wn2, bn2, wl, bl)


def kernel(in_feat, coord, edge_feat, edge_index, We1, be1, We2, be2, Wc1,
           bc1, Wc2, Wn1, bn1, Wn2, bn2, Wl, bl):
    srcw = edge_index[0].reshape(NW, CH, CB)
    dstw = edge_index[1].reshape(NW, CH, CB)
    dstw_s = edge_index[1].reshape(NW, CHS, CBS)

    ps, qd = _prep(in_feat, coord, We1[:F], We1[F:2 * F], be1.reshape(1, H))
    gsum = _gather(ps, qd, srcw, dstw)
    msg = _edge(gsum, edge_feat, We1[2 * F:2 * F + 1], We1[2 * F + 1:],
                We2, be2.reshape(1, H), Wc1, bc1.reshape(1, H), Wc2)
    zrows = jnp.zeros((NPAD, W), jnp.float32)
    acc = _scatter(msg, dstw_s, zrows)
    out = _node(in_feat, coord, acc[0, :N], acc[1, :N], Wn1[:F], Wn1[F:],
                bn1.reshape(1, H), Wn2, bn2.reshape(1, H), Wl,
                bl.reshape(1, -1))
    return out


# speedup vs baseline: 6.6539x; 1.0035x over previous
import functools

import jax
import jax.numpy as jnp
from jax import lax
from jax.experimental import pallas as pl
from jax.experimental.pallas import tpu as pltpu
from jax.experimental.pallas import tpu_sc as plsc

N = 10000
E = 320000
F = 128
H = 64
W = 128
NC, NS = 2, 16
NW = NC * NS
K = 5
ECH = E // K
EPWC = ECH // NW
CB = 80
CHC = EPWC // CB
RD = 5
CBS = 40
CHSC = EPWC // CBS
RDS = 2
NPAD = 10240
STR = NPAD // NS
EB = 2000


def _silu(x):
    return x / (1.0 + jnp.exp(-x))


def _prep_body(infeat_ref, coord_ref, whs_ref, whd_ref, be1_ref, ps_ref, qd_ref):
    x = infeat_ref[:]
    c = coord_ref[:]
    z = jnp.zeros((x.shape[0], W - H - 2), jnp.float32)
    p = jnp.dot(x, whs_ref[:], preferred_element_type=jnp.float32)
    q = jnp.dot(x, whd_ref[:], preferred_element_type=jnp.float32) + be1_ref[:]
    ps_ref[:] = jnp.concatenate([p, c, z], axis=1)
    qd_ref[:] = jnp.concatenate([q, -c, z], axis=1)


def _prep(in_feat, coord, whs, whd, be1):
    return pl.pallas_call(
        _prep_body,
        out_shape=[
            jax.ShapeDtypeStruct((N, W), jnp.float32),
            jax.ShapeDtypeStruct((N, W), jnp.float32),
        ],
    )(in_feat, coord, whs, whd, be1)


def _gather_body(ps, qd, srck, dstk, gsum_out, idx_s, idx_d, bufs,
                 sema, semb, semc):
    c = lax.axis_index("c")
    s = lax.axis_index("s")
    wid = s * NC + c
    pltpu.sync_copy(srck.at[wid], idx_s)
    pltpu.sync_copy(dstk.at[wid], idx_d)
    base = wid * EPWC

    def outer(o, carry):
        j0 = o * RD
        da = [pltpu.async_copy(ps.at[idx_s.at[j0 + r]], bufs.at[r], sema)
              for r in range(RD)]
        for d in da:
            d.wait()
        db = [pltpu.async_copy(qd.at[idx_d.at[j0 + r]], bufs.at[r], semb,
                               add=True) for r in range(RD)]
        for d in db:
            d.wait()
        dc = [pltpu.async_copy(bufs.at[r],
                               gsum_out.at[pl.ds(base + (j0 + r) * CB, CB)],
                               semc) for r in range(RD)]
        for d in dc:
            d.wait()
        return carry

    lax.fori_loop(0, CHC // RD, outer, 0)


def _gather(ps, qd, srck, dstk):
    fn = functools.partial(
        pl.kernel,
        out_type=jax.ShapeDtypeStruct((ECH, W), jnp.float32),
        mesh=plsc.VectorSubcoreMesh(core_axis_name="c", subcore_axis_name="s"),
        scratch_types=[
            pltpu.VMEM((CHC, CB), jnp.int32),
            pltpu.VMEM((CHC, CB), jnp.int32),
            pltpu.VMEM((RD, CB, W), jnp.float32),
            pltpu.SemaphoreType.DMA,
            pltpu.SemaphoreType.DMA,
            pltpu.SemaphoreType.DMA,
        ],
    )(_gather_body)
    return fn(ps, qd, srck, dstk)


def _edge_body(gsum, ef, wr, we, we2, be2, wc1, bc1, wc2, out):
    srow = gsum[:]
    t = srow[:, 0:H]
    d = srow[:, H:H + 2]
    radial = jnp.sum(d * d, axis=1, keepdims=True)
    xdn = d / (jnp.sqrt(radial) + 1e-30)
    t1 = t + radial * wr[:] + jnp.dot(ef[:], we[:],
                                      preferred_element_type=jnp.float32)
    m = _silu(t1)
    m = _silu(jnp.dot(m, we2[:], preferred_element_type=jnp.float32) + be2[:])
    cmid = _silu(jnp.dot(m, wc1[:], preferred_element_type=jnp.float32) + bc1[:])
    cg = jnp.dot(cmid, wc2[:], preferred_element_type=jnp.float32)
    pad = jnp.zeros((m.shape[0], W - H - 2), jnp.float32)
    out[:] = jnp.concatenate([m, cg * xdn, pad], axis=1)


def _edge(gsum, efk, wr, we, we2, be2, wc1, bc1, wc2):
    grid = (ECH // EB,)
    full = lambda shape: pl.BlockSpec(shape, lambda i: (0, 0))
    return pl.pallas_call(
        _edge_body,
        grid=grid,
        in_specs=[
            pl.BlockSpec((EB, W), lambda i: (i, 0)),
            pl.BlockSpec((EB, 16), lambda i: (i, 0)),
            full((1, H)),
            full((16, H)),
            full((H, H)),
            full((1, H)),
            full((H, H)),
            full((1, H)),
            full((H, 1)),
        ],
        out_specs=pl.BlockSpec((EB, W), lambda i: (i, 0)),
        out_shape=jax.ShapeDtypeStruct((ECH, W), jnp.float32),
    )(gsum, efk, wr, we, we2, be2, wc1, bc1, wc2)


def _scatter_body(m0, m1, m2, m3, m4, dstw, zrows, acc_out, idx_d, bufs,
                  sema, semb, acc_sh):
    msgs = [m0, m1, m2, m3, m4]
    c = lax.axis_index("c")
    s = lax.axis_index("s")
    wid = s * NC + c
    pltpu.sync_copy(dstw.at[wid], idx_d)
    pltpu.sync_copy(zrows.at[pl.ds(s * STR, STR)],
                    acc_sh.at[pl.ds(s * STR, STR)])
    plsc.subcore_barrier()
    base = wid * EPWC

    for kk in range(K):
        msg = msgs[kk]

        def outer(o, carry):
            j0 = o * RDS
            da = [pltpu.async_copy(msg.at[pl.ds(base + (j0 + r) * CBS, CBS)],
                                   bufs.at[r], sema) for r in range(RDS)]
            for d in da:
                d.wait()
            db = [pltpu.async_copy(bufs.at[r],
                                   acc_sh.at[idx_d.at[kk].at[j0 + r]], semb,
                                   add=True) for r in range(RDS)]
            for d in db:
                d.wait()
            return carry

        lax.fori_loop(0, CHSC // RDS, outer, 0)

    plsc.subcore_barrier()
    pltpu.sync_copy(acc_sh.at[pl.ds(s * STR, STR)],
                    acc_out.at[c].at[pl.ds(s * STR, STR)])


def _scatter(msgs, dstw, zrows):
    fn = functools.partial(
        pl.kernel,
        out_type=jax.ShapeDtypeStruct((NC, NPAD, W), jnp.float32),
        mesh=plsc.VectorSubcoreMesh(core_axis_name="c", subcore_axis_name="s"),
        scratch_types=[
            pltpu.VMEM((K, CHSC, CBS), jnp.int32),
            pltpu.VMEM((RDS, CBS, W), jnp.float32),
            pltpu.SemaphoreType.DMA,
            pltpu.SemaphoreType.DMA,
            pltpu.VMEM_SHARED((NPAD, W), jnp.float32),
        ],
    )(_scatter_body)
    return fn(*msgs, dstw, zrows)


def _node_body(infeat, coord, a0, a1, wn1a, wn1b, bn1, wn2, bn2, wl, bl, out):
    a = a0[:] + a1[:]
    hn = a[:, 0:H]
    xn = a[:, H:H + 2]
    h1 = _silu(jnp.dot(infeat[:], wn1a[:], preferred_element_type=jnp.float32)
               + jnp.dot(hn, wn1b[:], preferred_element_type=jnp.float32)
               + bn1[:])
    h2 = jnp.dot(h1, wn2[:], preferred_element_type=jnp.float32) + bn2[:]
    h3 = jnp.dot(h2, wl[:], preferred_element_type=jnp.float32) + bl[:]
    out[:] = jnp.concatenate([coord[:] + xn, h3], axis=1)


def _node(in_feat, coord, a0, a1, wn1a, wn1b, bn1, wn2, bn2, wl, bl):
    return pl.pallas_call(
        _node_body,
        out_shape=jax.ShapeDtypeStruct((N, 42), jnp.float32),
    )(in_feat, coord, a0, a1, wn1a, wn1b, bn1, wn2, bn2, wl, bl)


def kernel(in_feat, coord, edge_feat, edge_index, We1, be1, We2, be2, Wc1,
           bc1, Wc2, Wn1, bn1, Wn2, bn2, Wl, bl):
    srcw = edge_index[0].reshape(K, NW, CHC, CB)
    dstw = edge_index[1].reshape(K, NW, CHC, CB)
    dstw_s = edge_index[1].reshape(K, NW, CHSC, CBS).transpose(1, 0, 2, 3)
    efk = edge_feat.reshape(K, ECH, 16)

    ps, qd = _prep(in_feat, coord, We1[:F], We1[F:2 * F], be1.reshape(1, H))
    wr = We1[2 * F:2 * F + 1]
    we = We1[2 * F + 1:]
    msgs = []
    for k in range(K):
        gsum = _gather(ps, qd, srcw[k], dstw[k])
        msgs.append(_edge(gsum, efk[k], wr, we, We2, be2.reshape(1, H), Wc1,
                          bc1.reshape(1, H), Wc2))
    zrows = jnp.zeros((NPAD, W), jnp.float32)
    acc = _scatter(msgs, dstw_s, zrows)
    out = _node(in_feat, coord, acc[0, :N], acc[1, :N], Wn1[:F], Wn1[F:],
                bn1.reshape(1, H), Wn2, bn2.reshape(1, H), Wl,
                bl.reshape(1, -1))
    return out


# speedup vs baseline: 7.0050x; 1.0528x over previous
import functools

import jax
import jax.numpy as jnp
from jax import lax
from jax.experimental import pallas as pl
from jax.experimental.pallas import tpu as pltpu
from jax.experimental.pallas import tpu_sc as plsc

N = 10000
E = 320000
F = 128
H = 64
W = 128
NC, NS = 2, 16
NW = NC * NS
K = 5
ECH = E // K
EPWC = ECH // NW
CB = 80
CHC = EPWC // CB
RD = 5
CBS = 40
CHSC = EPWC // CBS
RDS = 2
NPAD = 10240
STR = NPAD // NS
EB = 2000


def _silu(x):
    return x / (1.0 + jnp.exp(-x))


def _prep_body(infeat_ref, coord_ref, whs_ref, whd_ref, be1_ref, ps_ref, qd_ref):
    x = infeat_ref[:]
    c = coord_ref[:]
    z = jnp.zeros((x.shape[0], W - H - 2), jnp.float32)
    p = jnp.dot(x, whs_ref[:], preferred_element_type=jnp.float32)
    q = jnp.dot(x, whd_ref[:], preferred_element_type=jnp.float32) + be1_ref[:]
    ps_ref[:] = jnp.concatenate([p, c, z], axis=1)
    qd_ref[:] = jnp.concatenate([q, -c, z], axis=1)


def _prep(in_feat, coord, whs, whd, be1):
    return pl.pallas_call(
        _prep_body,
        out_shape=[
            jax.ShapeDtypeStruct((N, W), jnp.float32),
            jax.ShapeDtypeStruct((N, W), jnp.float32),
        ],
    )(in_feat, coord, whs, whd, be1)


def _gather_body(ps, qd, srck, dstk, gsum_out, idx_s, idx_d, bufs,
                 sema, semb, semc):
    c = lax.axis_index("c")
    s = lax.axis_index("s")
    wid = s * NC + c
    pltpu.sync_copy(srck.at[wid], idx_s)
    pltpu.sync_copy(dstk.at[wid], idx_d)
    base = wid * EPWC

    def outer(o, carry):
        j0 = o * RD
        da = [pltpu.async_copy(ps.at[idx_s.at[j0 + r]], bufs.at[r], sema)
              for r in range(RD)]
        for d in da:
            d.wait()
        db = [pltpu.async_copy(qd.at[idx_d.at[j0 + r]], bufs.at[r], semb,
                               add=True) for r in range(RD)]
        for d in db:
            d.wait()
        dc = [pltpu.async_copy(bufs.at[r],
                               gsum_out.at[pl.ds(base + (j0 + r) * CB, CB)],
                               semc) for r in range(RD)]
        for d in dc:
            d.wait()
        return carry

    lax.fori_loop(0, CHC // RD, outer, 0)


def _gather(ps, qd, srck, dstk):
    fn = functools.partial(
        pl.kernel,
        out_type=jax.ShapeDtypeStruct((ECH, W), jnp.float32),
        mesh=plsc.VectorSubcoreMesh(core_axis_name="c", subcore_axis_name="s"),
        scratch_types=[
            pltpu.VMEM((CHC, CB), jnp.int32),
            pltpu.VMEM((CHC, CB), jnp.int32),
            pltpu.VMEM((RD, CB, W), jnp.float32),
            pltpu.SemaphoreType.DMA,
            pltpu.SemaphoreType.DMA,
            pltpu.SemaphoreType.DMA,
        ],
    )(_gather_body)
    return fn(ps, qd, srck, dstk)


def _edge_body(gsum, ef, wr, we, we2, be2, wc1, bc1, wc2, out):
    srow = gsum[:]
    t = srow[:, 0:H]
    d = srow[:, H:H + 2]
    radial = jnp.sum(d * d, axis=1, keepdims=True)
    xdn = d / (jnp.sqrt(radial) + 1e-30)
    t1 = t + radial * wr[:] + jnp.dot(ef[:], we[:],
                                      preferred_element_type=jnp.float32)
    m = _silu(t1)
    m = _silu(jnp.dot(m, we2[:], preferred_element_type=jnp.float32) + be2[:])
    cmid = _silu(jnp.dot(m, wc1[:], preferred_element_type=jnp.float32) + bc1[:])
    cg = jnp.dot(cmid, wc2[:], preferred_element_type=jnp.float32)
    pad = jnp.zeros((m.shape[0], W - H - 2), jnp.float32)
    out[:] = jnp.concatenate([m, cg * xdn, pad], axis=1)


def _edge(gsum, edge_feat, k, wr, we, we2, be2, wc1, bc1, wc2):
    grid = (ECH // EB,)
    koff = k * (ECH // EB)
    full = lambda shape: pl.BlockSpec(shape, lambda i: (0, 0))
    return pl.pallas_call(
        _edge_body,
        grid=grid,
        in_specs=[
            pl.BlockSpec((EB, W), lambda i: (i, 0)),
            pl.BlockSpec((EB, 16), lambda i: (i + koff, 0)),
            full((1, H)),
            full((16, H)),
            full((H, H)),
            full((1, H)),
            full((H, H)),
            full((1, H)),
            full((H, 1)),
        ],
        out_specs=pl.BlockSpec((EB, W), lambda i: (i, 0)),
        out_shape=jax.ShapeDtypeStruct((ECH, W), jnp.float32),
    )(gsum, edge_feat, wr, we, we2, be2, wc1, bc1, wc2)


def _scatter_body(m0, m1, m2, m3, m4, dstw, zrows, acc_out, idx_d, bufs,
                  sema, semb, acc_sh):
    msgs = [m0, m1, m2, m3, m4]
    c = lax.axis_index("c")
    s = lax.axis_index("s")
    wid = s * NC + c
    for kk in range(K):
        pltpu.sync_copy(dstw.at[kk].at[wid], idx_d.at[kk])
    pltpu.sync_copy(zrows.at[pl.ds(s * STR, STR)],
                    acc_sh.at[pl.ds(s * STR, STR)])
    plsc.subcore_barrier()
    base = wid * EPWC

    for kk in range(K):
        msg = msgs[kk]

        def outer(o, carry):
            j0 = o * RDS
            da = [pltpu.async_copy(msg.at[pl.ds(base + (j0 + r) * CBS, CBS)],
                                   bufs.at[r], sema) for r in range(RDS)]
            for d in da:
                d.wait()
            db = [pltpu.async_copy(bufs.at[r],
                                   acc_sh.at[idx_d.at[kk].at[j0 + r]], semb,
                                   add=True) for r in range(RDS)]
            for d in db:
                d.wait()
            return carry

        lax.fori_loop(0, CHSC // RDS, outer, 0)

    plsc.subcore_barrier()
    pltpu.sync_copy(acc_sh.at[pl.ds(s * STR, STR)],
                    acc_out.at[c].at[pl.ds(s * STR, STR)])


def _scatter(msgs, dstw, zrows):
    fn = functools.partial(
        pl.kernel,
        out_type=jax.ShapeDtypeStruct((NC, NPAD, W), jnp.float32),
        mesh=plsc.VectorSubcoreMesh(core_axis_name="c", subcore_axis_name="s"),
        scratch_types=[
            pltpu.VMEM((K, CHSC, CBS), jnp.int32),
            pltpu.VMEM((RDS, CBS, W), jnp.float32),
            pltpu.SemaphoreType.DMA,
            pltpu.SemaphoreType.DMA,
            pltpu.VMEM_SHARED((NPAD, W), jnp.float32),
        ],
    )(_scatter_body)
    return fn(*msgs, dstw, zrows)


def _node_body(infeat, coord, a0, a1, wn1a, wn1b, bn1, wn2, bn2, wl, bl, out):
    a = a0[:] + a1[:]
    hn = a[:, 0:H]
    xn = a[:, H:H + 2]
    h1 = _silu(jnp.dot(infeat[:], wn1a[:], preferred_element_type=jnp.float32)
               + jnp.dot(hn, wn1b[:], preferred_element_type=jnp.float32)
               + bn1[:])
    h2 = jnp.dot(h1, wn2[:], preferred_element_type=jnp.float32) + bn2[:]
    h3 = jnp.dot(h2, wl[:], preferred_element_type=jnp.float32) + bl[:]
    out[:] = jnp.concatenate([coord[:] + xn, h3], axis=1)


def _node(in_feat, coord, a0, a1, wn1a, wn1b, bn1, wn2, bn2, wl, bl):
    return pl.pallas_call(
        _node_body,
        out_shape=jax.ShapeDtypeStruct((N, 42), jnp.float32),
    )(in_feat, coord, a0, a1, wn1a, wn1b, bn1, wn2, bn2, wl, bl)


def kernel(in_feat, coord, edge_feat, edge_index, We1, be1, We2, be2, Wc1,
           bc1, Wc2, Wn1, bn1, Wn2, bn2, Wl, bl):
    srcw = edge_index[0].reshape(K, NW, CHC, CB)
    dstw = edge_index[1].reshape(K, NW, CHC, CB)
    dstw_s = edge_index[1].reshape(K, NW, CHSC, CBS)

    ps, qd = _prep(in_feat, coord, We1[:F], We1[F:2 * F], be1.reshape(1, H))
    wr = We1[2 * F:2 * F + 1]
    we = We1[2 * F + 1:]
    msgs = []
    for k in range(K):
        gsum = _gather(ps, qd, srcw[k], dstw[k])
        msgs.append(_edge(gsum, edge_feat, k, wr, we, We2, be2.reshape(1, H),
                          Wc1, bc1.reshape(1, H), Wc2))
    zrows = jnp.zeros((NPAD, W), jnp.float32)
    acc = _scatter(msgs, dstw_s, zrows)
    out = _node(in_feat, coord, acc[0, :N], acc[1, :N], Wn1[:F], Wn1[F:],
                bn1.reshape(1, H), Wn2, bn2.reshape(1, H), Wl,
                bl.reshape(1, -1))
    return out


# speedup vs baseline: 7.7896x; 1.1120x over previous
import functools

import jax
import jax.numpy as jnp
from jax import lax
from jax.experimental import pallas as pl
from jax.experimental.pallas import tpu as pltpu
from jax.experimental.pallas import tpu_sc as plsc

N = 10000
E = 320000
F = 128
H = 64
W = 128
NC, NS = 2, 16
NW = NC * NS
K = 5
ECH = E // K
EPWC = ECH // NW
CB = 80
CHC = EPWC // CB
RD = 5
CBS = 40
CHSC = EPWC // CBS
RDS = 2
NPAD = 10240
STR = NPAD // NS
EB = 2000


def _silu(x):
    return x / (1.0 + jnp.exp(-x))


def _prep_body(infeat_ref, coord_ref, whs_ref, whd_ref, be1_ref, ps_ref, qd_ref):
    x = infeat_ref[:]
    c = coord_ref[:]
    z = jnp.zeros((x.shape[0], W - H - 2), jnp.float32)
    p = jnp.dot(x, whs_ref[:], preferred_element_type=jnp.float32)
    q = jnp.dot(x, whd_ref[:], preferred_element_type=jnp.float32) + be1_ref[:]
    ps_ref[:] = jnp.concatenate([p, c, z], axis=1)
    qd_ref[:] = jnp.concatenate([q, -c, z], axis=1)


def _prep(in_feat, coord, whs, whd, be1):
    return pl.pallas_call(
        _prep_body,
        out_shape=[
            jax.ShapeDtypeStruct((N, W), jnp.float32),
            jax.ShapeDtypeStruct((N, W), jnp.float32),
        ],
    )(in_feat, coord, whs, whd, be1)


def _gather_body(ps, qd, srck, dstk, gsum_out, idx_s, idx_d, bufs,
                 sema, semb, semc):
    c = lax.axis_index("c")
    s = lax.axis_index("s")
    wid = s * NC + c
    pltpu.sync_copy(srck.at[wid], idx_s)
    pltpu.sync_copy(dstk.at[wid], idx_d)
    base = wid * EPWC

    def outer(o, carry):
        j0 = o * RD
        da = [pltpu.async_copy(ps.at[idx_s.at[j0 + r]], bufs.at[r], sema)
              for r in range(RD)]
        for d in da:
            d.wait()
        db = [pltpu.async_copy(qd.at[idx_d.at[j0 + r]], bufs.at[r], semb,
                               add=True) for r in range(RD)]
        for d in db:
            d.wait()
        dc = [pltpu.async_copy(bufs.at[r],
                               gsum_out.at[pl.ds(base + (j0 + r) * CB, CB)],
                               semc) for r in range(RD)]
        for d in dc:
            d.wait()
        return carry

    lax.fori_loop(0, CHC // RD, outer, 0)


def _gather(ps, qd, srck, dstk):
    fn = functools.partial(
        pl.kernel,
        out_type=jax.ShapeDtypeStruct((ECH, W), jnp.float32),
        mesh=plsc.VectorSubcoreMesh(core_axis_name="c", subcore_axis_name="s"),
        scratch_types=[
            pltpu.VMEM((CHC, CB), jnp.int32),
            pltpu.VMEM((CHC, CB), jnp.int32),
            pltpu.VMEM((RD, CB, W), jnp.float32),
            pltpu.SemaphoreType.DMA,
            pltpu.SemaphoreType.DMA,
            pltpu.SemaphoreType.DMA,
        ],
    )(_gather_body)
    return fn(ps, qd, srck, dstk)


def _edge_body(gsum, ef, wr, we, we2, be2, wc1, bc1, wc2, out):
    srow = gsum[:]
    t = srow[:, 0:H]
    d = srow[:, H:H + 2]
    radial = jnp.sum(d * d, axis=1, keepdims=True)
    xdn = d / (jnp.sqrt(radial) + 1e-30)
    t1 = t + radial * wr[:] + jnp.dot(ef[:], we[:],
                                      preferred_element_type=jnp.float32)
    m = _silu(t1)
    m = _silu(jnp.dot(m, we2[:], preferred_element_type=jnp.float32) + be2[:])
    cmid = _silu(jnp.dot(m, wc1[:], preferred_element_type=jnp.float32) + bc1[:])
    cg = jnp.dot(cmid, wc2[:], preferred_element_type=jnp.float32)
    pad = jnp.zeros((m.shape[0], W - H - 2), jnp.float32)
    out[:] = jnp.concatenate([m, cg * xdn, pad], axis=1)


def _edge(gsum, edge_feat, k, wr, we, we2, be2, wc1, bc1, wc2):
    grid = (ECH // EB,)
    koff = k * (ECH // EB)
    full = lambda shape: pl.BlockSpec(shape, lambda i: (0, 0))
    return pl.pallas_call(
        _edge_body,
        grid=grid,
        in_specs=[
            pl.BlockSpec((EB, W), lambda i: (i, 0)),
            pl.BlockSpec((EB, 16), lambda i: (i + koff, 0)),
            full((1, H)),
            full((16, H)),
            full((H, H)),
            full((1, H)),
            full((H, H)),
            full((1, H)),
            full((H, 1)),
        ],
        out_specs=pl.BlockSpec((EB, W), lambda i: (i, 0)),
        out_shape=jax.ShapeDtypeStruct((ECH, W), jnp.float32),
    )(gsum, edge_feat, wr, we, we2, be2, wc1, bc1, wc2)


def _scatter_group(msgs, ks, dstw, zrows):
    nk = len(ks)

    def body(*args):
        msg_refs = args[:nk]
        dstw_ref, zrows_ref, acc_out, idx_d, bufs, sema, semb, acc_sh = args[nk:]
        c = lax.axis_index("c")
        s = lax.axis_index("s")
        wid = s * NC + c
        for i, kk in enumerate(ks):
            pltpu.sync_copy(dstw_ref.at[kk].at[wid], idx_d.at[i])
        pltpu.sync_copy(zrows_ref.at[pl.ds(s * STR, STR)],
                        acc_sh.at[pl.ds(s * STR, STR)])
        plsc.subcore_barrier()
        base = wid * EPWC

        for i in range(nk):
            msg = msg_refs[i]

            def outer(o, carry):
                j0 = o * RDS
                da = [pltpu.async_copy(
                    msg.at[pl.ds(base + (j0 + r) * CBS, CBS)],
                    bufs.at[r], sema) for r in range(RDS)]
                for d in da:
                    d.wait()
                db = [pltpu.async_copy(bufs.at[r],
                                       acc_sh.at[idx_d.at[i].at[j0 + r]],
                                       semb, add=True) for r in range(RDS)]
                for d in db:
                    d.wait()
                return carry

            lax.fori_loop(0, CHSC // RDS, outer, 0)

        plsc.subcore_barrier()
        pltpu.sync_copy(acc_sh.at[pl.ds(s * STR, STR)],
                        acc_out.at[c].at[pl.ds(s * STR, STR)])

    fn = functools.partial(
        pl.kernel,
        out_type=jax.ShapeDtypeStruct((NC, NPAD, W), jnp.float32),
        mesh=plsc.VectorSubcoreMesh(core_axis_name="c", subcore_axis_name="s"),
        scratch_types=[
            pltpu.VMEM((nk, CHSC, CBS), jnp.int32),
            pltpu.VMEM((RDS, CBS, W), jnp.float32),
            pltpu.SemaphoreType.DMA,
            pltpu.SemaphoreType.DMA,
            pltpu.VMEM_SHARED((NPAD, W), jnp.float32),
        ],
    )(body)
    return fn(*msgs, dstw, zrows)


def _node_body(infeat, coord, a0, a1, a2, a3, wn1a, wn1b, bn1, wn2, bn2, wl,
               bl, out):
    a = (a0[:] + a1[:]) + (a2[:] + a3[:])
    hn = a[:, 0:H]
    xn = a[:, H:H + 2]
    h1 = _silu(jnp.dot(infeat[:], wn1a[:], preferred_element_type=jnp.float32)
               + jnp.dot(hn, wn1b[:], preferred_element_type=jnp.float32)
               + bn1[:])
    h2 = jnp.dot(h1, wn2[:], preferred_element_type=jnp.float32) + bn2[:]
    h3 = jnp.dot(h2, wl[:], preferred_element_type=jnp.float32) + bl[:]
    out[:] = jnp.concatenate([coord[:] + xn, h3], axis=1)


def _node(in_feat, coord, accs, wn1a, wn1b, bn1, wn2, bn2, wl, bl):
    return pl.pallas_call(
        _node_body,
        out_shape=jax.ShapeDtypeStruct((N, 42), jnp.float32),
    )(in_feat, coord, *accs, wn1a, wn1b, bn1, wn2, bn2, wl, bl)


def kernel(in_feat, coord, edge_feat, edge_index, We1, be1, We2, be2, Wc1,
           bc1, Wc2, Wn1, bn1, Wn2, bn2, Wl, bl):
    srcw = edge_index[0].reshape(K, NW, CHC, CB)
    dstw = edge_index[1].reshape(K, NW, CHC, CB)
    dstw_s = edge_index[1].reshape(K, NW, CHSC, CBS)

    ps, qd = _prep(in_feat, coord, We1[:F], We1[F:2 * F], be1.reshape(1, H))
    wr = We1[2 * F:2 * F + 1]
    we = We1[2 * F + 1:]
    msgs = []
    for k in range(K):
        gsum = _gather(ps, qd, srcw[k], dstw[k])
        msgs.append(_edge(gsum, edge_feat, k, wr, we, We2, be2.reshape(1, H),
                          Wc1, bc1.reshape(1, H), Wc2))
    zrows = jnp.zeros((NPAD, W), jnp.float32)
    acc_a = _scatter_group(msgs[:3], [0, 1, 2], dstw_s, zrows)
    acc_b = _scatter_group(msgs[3:], [3, 4], dstw_s, zrows)
    accs = [acc_a[0, :N], acc_a[1, :N], acc_b[0, :N], acc_b[1, :N]]
    out = _node(in_feat, coord, accs, Wn1[:F], Wn1[F:],
                bn1.reshape(1, H), Wn2, bn2.reshape(1, H), Wl,
                bl.reshape(1, -1))
    return out


# speedup vs baseline: 9.3379x; 1.1988x over previous
import functools

import jax
import jax.numpy as jnp
from jax import lax
from jax.experimental import pallas as pl
from jax.experimental.pallas import tpu as pltpu
from jax.experimental.pallas import tpu_sc as plsc

N = 10000
E = 320000
F = 128
H = 64
W = 128
NC, NS = 2, 16
NW = NC * NS
K = 5
ECH = E // K
EPWC = ECH // NW
CB = 80
CHC = EPWC // CB
RD = 5
RDS = 2
NPAD = 10112
STR = NPAD // NS
EB = 3200


def _silu(x):
    return x * (0.5 * jnp.tanh(0.5 * x) + 0.5)


def _prep_body(infeat_ref, coord_ref, whs_ref, whd_ref, be1_ref, ps_ref, qd_ref):
    x = infeat_ref[:]
    c = coord_ref[:]
    z = jnp.zeros((x.shape[0], W - H - 2), jnp.float32)
    p = jnp.dot(x, whs_ref[:], preferred_element_type=jnp.float32)
    q = jnp.dot(x, whd_ref[:], preferred_element_type=jnp.float32) + be1_ref[:]
    ps_ref[:] = jnp.concatenate([p, c, z], axis=1)
    qd_ref[:] = jnp.concatenate([q, -c, z], axis=1)


def _prep(in_feat, coord, whs, whd, be1):
    return pl.pallas_call(
        _prep_body,
        out_shape=[
            jax.ShapeDtypeStruct((N, W), jnp.float32),
            jax.ShapeDtypeStruct((N, W), jnp.float32),
        ],
    )(in_feat, coord, whs, whd, be1)


def _gather_body(ps, qd, srck, dstk, gsum_out, idx_s, idx_d, bufs,
                 sema, semb, semc):
    c = lax.axis_index("c")
    s = lax.axis_index("s")
    wid = s * NC + c
    pltpu.sync_copy(srck.at[wid], idx_s)
    pltpu.sync_copy(dstk.at[wid], idx_d)
    base = wid * EPWC
    ngrp = CHC // RD

    def fire_a(g, grp):
        return [pltpu.async_copy(ps.at[idx_s.at[g * RD + r]],
                                 bufs.at[grp].at[r], sema) for r in range(RD)]

    pend_c = [None, None]
    da = fire_a(0, 0)
    for g in range(ngrp):
        grp = g % 2
        for d in da:
            d.wait()
        db = [pltpu.async_copy(qd.at[idx_d.at[g * RD + r]],
                               bufs.at[grp].at[r], semb, add=True)
              for r in range(RD)]
        for d in db:
            d.wait()
        pend_c[grp] = [
            pltpu.async_copy(bufs.at[grp].at[r],
                             gsum_out.at[pl.ds(base + (g * RD + r) * CB, CB)],
                             semc) for r in range(RD)]
        if g + 1 < ngrp:
            nxt = 1 - grp
            if pend_c[nxt] is not None:
                for d in pend_c[nxt]:
                    d.wait()
                pend_c[nxt] = None
            da = fire_a(g + 1, nxt)
    for dcl in pend_c:
        if dcl is not None:
            for d in dcl:
                d.wait()


def _gather(ps, qd, srck, dstk):
    fn = functools.partial(
        pl.kernel,
        out_type=jax.ShapeDtypeStruct((ECH, W), jnp.float32),
        mesh=plsc.VectorSubcoreMesh(core_axis_name="c", subcore_axis_name="s"),
        scratch_types=[
            pltpu.VMEM((CHC, CB), jnp.int32),
            pltpu.VMEM((CHC, CB), jnp.int32),
            pltpu.VMEM((2, RD, CB, W), jnp.float32),
            pltpu.SemaphoreType.DMA,
            pltpu.SemaphoreType.DMA,
            pltpu.SemaphoreType.DMA,
        ],
    )(_gather_body)
    return fn(ps, qd, srck, dstk)


def _edge_body(gsum, eft, wr, we, we2, be2, wc1, bc1, wc2, out):
    srow = gsum[:]
    t = srow[:, 0:H]
    d = srow[:, H:H + 2]
    radial = jnp.sum(d * d, axis=1, keepdims=True)
    xdn = d / (jnp.sqrt(radial) + 1e-30)
    efw = lax.dot_general(eft[:], we[:], (((0,), (0,)), ((), ())),
                          preferred_element_type=jnp.float32)
    t1 = t + radial * wr[:] + efw
    m = _silu(t1)
    m = _silu(jnp.dot(m, we2[:], preferred_element_type=jnp.float32) + be2[:])
    cmid = _silu(jnp.dot(m, wc1[:], preferred_element_type=jnp.float32) + bc1[:])
    cg = jnp.dot(cmid, wc2[:], preferred_element_type=jnp.float32)
    pad = jnp.zeros((m.shape[0], W - H - 2), jnp.float32)
    out[:] = jnp.concatenate([m, cg * xdn, pad], axis=1)


def _edge(gsum, eft, k, wr, we, we2, be2, wc1, bc1, wc2):
    grid = (ECH // EB,)
    koff = k * (ECH // EB)
    full = lambda shape: pl.BlockSpec(shape, lambda i: (0, 0))
    return pl.pallas_call(
        _edge_body,
        grid=grid,
        in_specs=[
            pl.BlockSpec((EB, W), lambda i: (i, 0)),
            pl.BlockSpec((16, EB), lambda i: (0, i + koff)),
            full((1, H)),
            full((16, H)),
            full((H, H)),
            full((1, H)),
            full((H, H)),
            full((1, H)),
            full((H, 1)),
        ],
        out_specs=pl.BlockSpec((EB, W), lambda i: (i, 0)),
        out_shape=jax.ShapeDtypeStruct((ECH, W), jnp.float32),
    )(gsum, eft, wr, we, we2, be2, wc1, bc1, wc2)


def _scatter_group(msgs, ks, dstw, zrows):
    nk = len(ks)

    def body(*args):
        msg_refs = args[:nk]
        dstw_ref, zrows_ref, acc_out, idx_d, bufs, sema, semb, acc_sh = args[nk:]
        c = lax.axis_index("c")
        s = lax.axis_index("s")
        wid = s * NC + c
        for i, kk in enumerate(ks):
            pltpu.sync_copy(dstw_ref.at[kk].at[wid], idx_d.at[i])
        pltpu.sync_copy(zrows_ref.at[pl.ds(s * STR, STR)],
                        acc_sh.at[pl.ds(s * STR, STR)])
        plsc.subcore_barrier()
        base = wid * EPWC

        for i in range(nk):
            msg = msg_refs[i]

            def pair(j0):
                da = [pltpu.async_copy(
                    msg.at[pl.ds(base + (j0 + r) * CB, CB)],
                    bufs.at[r], sema) for r in range(RDS)]
                for d in da:
                    d.wait()
                db = [pltpu.async_copy(bufs.at[r],
                                       acc_sh.at[idx_d.at[i].at[j0 + r]],
                                       semb, add=True) for r in range(RDS)]
                for d in db:
                    d.wait()

            def outer(o, carry):
                pair(o * RDS)
                return carry

            lax.fori_loop(0, (CHC - 1) // RDS, outer, 0)
            pltpu.async_copy(msg.at[pl.ds(base + (CHC - 1) * CB, CB)],
                             bufs.at[0], sema).wait()
            pltpu.async_copy(bufs.at[0], acc_sh.at[idx_d.at[i].at[CHC - 1]],
                             semb, add=True).wait()

        plsc.subcore_barrier()
        pltpu.sync_copy(acc_sh.at[pl.ds(s * STR, STR)],
                        acc_out.at[c].at[pl.ds(s * STR, STR)])

    fn = functools.partial(
        pl.kernel,
        out_type=jax.ShapeDtypeStruct((NC, NPAD, W), jnp.float32),
        mesh=plsc.VectorSubcoreMesh(core_axis_name="c", subcore_axis_name="s"),
        scratch_types=[
            pltpu.VMEM((nk, CHC, CB), jnp.int32),
            pltpu.VMEM((RDS, CB, W), jnp.float32),
            pltpu.SemaphoreType.DMA,
            pltpu.SemaphoreType.DMA,
            pltpu.VMEM_SHARED((NPAD, W), jnp.float32),
        ],
    )(body)
    return fn(*msgs, dstw, zrows)


def _node_body(infeat, coord, a0, a1, a2, a3, a4, a5, wn1a, wn1b, bn1, wn2,
               bn2, wl, bl, out):
    a = (a0[:] + a1[:]) + (a2[:] + a3[:]) + (a4[:] + a5[:])
    hn = a[:, 0:H]
    xn = a[:, H:H + 2]
    h1 = _silu(jnp.dot(infeat[:], wn1a[:], preferred_element_type=jnp.float32)
               + jnp.dot(hn, wn1b[:], preferred_element_type=jnp.float32)
               + bn1[:])
    h2 = jnp.dot(h1, wn2[:], preferred_element_type=jnp.float32) + bn2[:]
    h3 = jnp.dot(h2, wl[:], preferred_element_type=jnp.float32) + bl[:]
    out[:] = jnp.concatenate([coord[:] + xn, h3], axis=1)


def _node(in_feat, coord, accs, wn1a, wn1b, bn1, wn2, bn2, wl, bl):
    return pl.pallas_call(
        _node_body,
        out_shape=jax.ShapeDtypeStruct((N, 42), jnp.float32),
    )(in_feat, coord, *accs, wn1a, wn1b, bn1, wn2, bn2, wl, bl)


def kernel(in_feat, coord, edge_feat, edge_index, We1, be1, We2, be2, Wc1,
           bc1, Wc2, Wn1, bn1, Wn2, bn2, Wl, bl):
    srcw = edge_index[0].reshape(K, NW, CHC, CB)
    dstw = edge_index[1].reshape(K, NW, CHC, CB)

    ps, qd = _prep(in_feat, coord, We1[:F], We1[F:2 * F], be1.reshape(1, H))
    wr = We1[2 * F:2 * F + 1]
    we = We1[2 * F + 1:]
    eft = edge_feat.T
    msgs = []
    for k in range(K):
        gsum = _gather(ps, qd, srcw[k], dstw[k])
        msgs.append(_edge(gsum, eft, k, wr, we, We2, be2.reshape(1, H),
                          Wc1, bc1.reshape(1, H), Wc2))
    zrows = jnp.zeros((NPAD, W), jnp.float32)
    acc_a = _scatter_group(msgs[:2], [0, 1], dstw, zrows)
    acc_b = _scatter_group(msgs[2:4], [2, 3], dstw, zrows)
    acc_c = _scatter_group(msgs[4:], [4], dstw, zrows)
    accs = [acc_a[0, :N], acc_a[1, :N], acc_b[0, :N], acc_b[1, :N],
            acc_c[0, :N], acc_c[1, :N]]
    out = _node(in_feat, coord, accs, Wn1[:F], Wn1[F:],
                bn1.reshape(1, H), Wn2, bn2.reshape(1, H), Wl,
                bl.reshape(1, -1))
    return out


# speedup vs baseline: 10.0199x; 1.0730x over previous
import functools

import jax
import jax.numpy as jnp
from jax import lax
from jax.experimental import pallas as pl
from jax.experimental.pallas import tpu as pltpu
from jax.experimental.pallas import tpu_sc as plsc

N = 10000
E = 320000
F = 128
H = 64
W = 128
NC, NS = 2, 16
NW = NC * NS
K = 5
ECH = E // K
EPWC = ECH // NW
CB = 80
CHC = EPWC // CB
RD = 5
RDS = 2
NPAD = 10112
STR = NPAD // NS
EB = 3200


def _silu(x):
    return x * (0.5 * jnp.tanh(0.5 * x) + 0.5)


def _prep_body(infeat_ref, coord_ref, whs_ref, whd_ref, be1_ref, ps_ref, qd_ref):
    x = infeat_ref[:]
    c = coord_ref[:]
    z = jnp.zeros((x.shape[0], W - H - 2), jnp.float32)
    p = jnp.dot(x, whs_ref[:], preferred_element_type=jnp.float32)
    q = jnp.dot(x, whd_ref[:], preferred_element_type=jnp.float32) + be1_ref[:]
    ps_ref[:] = jnp.concatenate([p, c, z], axis=1)
    qd_ref[:] = jnp.concatenate([q, -c, z], axis=1)


def _prep(in_feat, coord, whs, whd, be1):
    return pl.pallas_call(
        _prep_body,
        out_shape=[
            jax.ShapeDtypeStruct((N, W), jnp.float32),
            jax.ShapeDtypeStruct((N, W), jnp.float32),
        ],
    )(in_feat, coord, whs, whd, be1)


def _gather_body(ps, qd, srck, dstk, gsum_out, idx_s, idx_d, bufs,
                 sema, semb, semc):
    c = lax.axis_index("c")
    s = lax.axis_index("s")
    wid = s * NC + c
    pltpu.sync_copy(srck.at[wid], idx_s)
    pltpu.sync_copy(dstk.at[wid], idx_d)
    base = wid * EPWC

    def outer(o, carry):
        j0 = o * RD
        da = [pltpu.async_copy(ps.at[idx_s.at[j0 + r]], bufs.at[r], sema)
              for r in range(RD)]
        for d in da:
            d.wait()
        db = [pltpu.async_copy(qd.at[idx_d.at[j0 + r]], bufs.at[r], semb,
                               add=True) for r in range(RD)]
        for d in db:
            d.wait()
        dc = [pltpu.async_copy(bufs.at[r],
                               gsum_out.at[pl.ds(base + (j0 + r) * CB, CB)],
                               semc) for r in range(RD)]
        for d in dc:
            d.wait()
        return carry

    lax.fori_loop(0, CHC // RD, outer, 0)


def _gather(ps, qd, srck, dstk):
    fn = functools.partial(
        pl.kernel,
        out_type=jax.ShapeDtypeStruct((ECH, W), jnp.float32),
        mesh=plsc.VectorSubcoreMesh(core_axis_name="c", subcore_axis_name="s"),
        scratch_types=[
            pltpu.VMEM((CHC, CB), jnp.int32),
            pltpu.VMEM((CHC, CB), jnp.int32),
            pltpu.VMEM((RD, CB, W), jnp.float32),
            pltpu.SemaphoreType.DMA,
            pltpu.SemaphoreType.DMA,
            pltpu.SemaphoreType.DMA,
        ],
    )(_gather_body)
    return fn(ps, qd, srck, dstk)


def _edge_body(gsum, eft, wr, we, we2, be2, wc1, bc1, wc2, out):
    srow = gsum[:]
    t = srow[:, 0:H]
    d = srow[:, H:H + 2]
    radial = jnp.sum(d * d, axis=1, keepdims=True)
    xdn = d / (jnp.sqrt(radial) + 1e-30)
    efw = lax.dot_general(eft[:], we[:], (((0,), (0,)), ((), ())),
                          preferred_element_type=jnp.float32)
    t1 = t + radial * wr[:] + efw
    m = _silu(t1)
    m = _silu(jnp.dot(m, we2[:], preferred_element_type=jnp.float32) + be2[:])
    cmid = _silu(jnp.dot(m, wc1[:], preferred_element_type=jnp.float32) + bc1[:])
    cg = jnp.dot(cmid, wc2[:], preferred_element_type=jnp.float32)
    pad = jnp.zeros((m.shape[0], W - H - 2), jnp.float32)
    out[:] = jnp.concatenate([m, cg * xdn, pad], axis=1)


def _edge(gsum, eft, k, wr, we, we2, be2, wc1, bc1, wc2):
    grid = (ECH // EB,)
    koff = k * (ECH // EB)
    full = lambda shape: pl.BlockSpec(shape, lambda i: (0, 0))
    return pl.pallas_call(
        _edge_body,
        grid=grid,
        in_specs=[
            pl.BlockSpec((EB, W), lambda i: (i, 0)),
            pl.BlockSpec((16, EB), lambda i: (0, i + koff)),
            full((1, H)),
            full((16, H)),
            full((H, H)),
            full((1, H)),
            full((H, H)),
            full((1, H)),
            full((H, 1)),
        ],
        out_specs=pl.BlockSpec((EB, W), lambda i: (i, 0)),
        out_shape=jax.ShapeDtypeStruct((ECH, W), jnp.float32),
    )(gsum, eft, wr, we, we2, be2, wc1, bc1, wc2)


def _scatter_group(msgs, ks, dstw, zrows):
    nk = len(ks)

    def body(*args):
        msg_refs = args[:nk]
        dstw_ref, zrows_ref, acc_out, idx_d, bufs, sema, semb, acc_sh = args[nk:]
        c = lax.axis_index("c")
        s = lax.axis_index("s")
        wid = s * NC + c
        for i, kk in enumerate(ks):
            pltpu.sync_copy(dstw_ref.at[kk].at[wid], idx_d.at[i])
        pltpu.sync_copy(zrows_ref.at[pl.ds(s * STR, STR)],
                        acc_sh.at[pl.ds(s * STR, STR)])
        plsc.subcore_barrier()
        base = wid * EPWC

        for i in range(nk):
            msg = msg_refs[i]

            def pair(j0):
                da = [pltpu.async_copy(
                    msg.at[pl.ds(base + (j0 + r) * CB, CB)],
                    bufs.at[r], sema) for r in range(RDS)]
                for d in da:
                    d.wait()
                db = [pltpu.async_copy(bufs.at[r],
                                       acc_sh.at[idx_d.at[i].at[j0 + r]],
                                       semb, add=True) for r in range(RDS)]
                for d in db:
                    d.wait()

            def outer(o, carry):
                pair(o * RDS)
                return carry

            lax.fori_loop(0, (CHC - 1) // RDS, outer, 0)
            pltpu.async_copy(msg.at[pl.ds(base + (CHC - 1) * CB, CB)],
                             bufs.at[0], sema).wait()
            pltpu.async_copy(bufs.at[0], acc_sh.at[idx_d.at[i].at[CHC - 1]],
                             semb, add=True).wait()

        plsc.subcore_barrier()
        pltpu.sync_copy(acc_sh.at[pl.ds(s * STR, STR)],
                        acc_out.at[c].at[pl.ds(s * STR, STR)])

    fn = functools.partial(
        pl.kernel,
        out_type=jax.ShapeDtypeStruct((NC, NPAD, W), jnp.float32),
        mesh=plsc.VectorSubcoreMesh(core_axis_name="c", subcore_axis_name="s"),
        scratch_types=[
            pltpu.VMEM((nk, CHC, CB), jnp.int32),
            pltpu.VMEM((RDS, CB, W), jnp.float32),
            pltpu.SemaphoreType.DMA,
            pltpu.SemaphoreType.DMA,
            pltpu.VMEM_SHARED((NPAD, W), jnp.float32),
        ],
    )(body)
    return fn(*msgs, dstw, zrows)


def _node_body(infeat, coord, a0, a1, a2, a3, wn1a, wn1b, bn1, wn2, bn2, wl,
               bl, out):
    a = (a0[:] + a1[:]) + (a2[:] + a3[:])
    hn = a[:, 0:H]
    xn = a[:, H:H + 2]
    h1 = _silu(jnp.dot(infeat[:], wn1a[:], preferred_element_type=jnp.float32)
               + jnp.dot(hn, wn1b[:], preferred_element_type=jnp.float32)
               + bn1[:])
    h2 = jnp.dot(h1, wn2[:], preferred_element_type=jnp.float32) + bn2[:]
    h3 = jnp.dot(h2, wl[:], preferred_element_type=jnp.float32) + bl[:]
    out[:] = jnp.concatenate([coord[:] + xn, h3], axis=1)


def _node(in_feat, coord, accs, wn1a, wn1b, bn1, wn2, bn2, wl, bl):
    return pl.pallas_call(
        _node_body,
        out_shape=jax.ShapeDtypeStruct((N, 42), jnp.float32),
    )(in_feat, coord, *accs, wn1a, wn1b, bn1, wn2, bn2, wl, bl)


def kernel(in_feat, coord, edge_feat, edge_index, We1, be1, We2, be2, Wc1,
           bc1, Wc2, Wn1, bn1, Wn2, bn2, Wl, bl):
    srcw = edge_index[0].reshape(K, NW, CHC, CB)
    dstw = edge_index[1].reshape(K, NW, CHC, CB)

    ps, qd = _prep(in_feat, coord, We1[:F], We1[F:2 * F], be1.reshape(1, H))
    wr = We1[2 * F:2 * F + 1]
    we = We1[2 * F + 1:]
    eft = edge_feat.T
    msgs = []
    for k in range(K):
        gsum = _gather(ps, qd, srcw[k], dstw[k])
        msgs.append(_edge(gsum, eft, k, wr, we, We2, be2.reshape(1, H),
                          Wc1, bc1.reshape(1, H), Wc2))
    zrows = jnp.zeros((NPAD, W), jnp.float32)
    acc_a = _scatter_group(msgs[:3], [0, 1, 2], dstw, zrows)
    acc_b = _scatter_group(msgs[3:], [3, 4], dstw, zrows)
    accs = [acc_a[0, :N], acc_a[1, :N], acc_b[0, :N], acc_b[1, :N]]
    out = _node(in_feat, coord, accs, Wn1[:F], Wn1[F:],
                bn1.reshape(1, H), Wn2, bn2.reshape(1, H), Wl,
                bl.reshape(1, -1))
    return out
